# baseline (speedup 1.0000x reference)
import functools

import jax
import jax.numpy as jnp
from jax import lax
from jax.experimental import pallas as pl
from jax.experimental.pallas import tpu as pltpu
from jax.experimental.pallas import tpu_sc as plsc

B = 1024
N_REL = 237
HIDDEN = 64
NS = 8
N_ENT = 100000
N_EDGE = 3200000

NW = 32
CB = B // NW
CE = CB * 2


def _sc_gather_chain():
    mesh = plsc.VectorSubcoreMesh(core_axis_name="c", subcore_axis_name="s")

    n1 = CE * NS
    n2 = n1 * 2 * NS
    L = 16

    @functools.partial(
        pl.kernel,
        out_type=[
            jax.ShapeDtypeStruct((NW, n1), jnp.int32),
            jax.ShapeDtypeStruct((NW, n1), jnp.int32),
            jax.ShapeDtypeStruct((NW, n2), jnp.int32),
            jax.ShapeDtypeStruct((NW, n2), jnp.int32),
        ],
        mesh=mesh,
        compiler_params=pltpu.CompilerParams(needs_layout_passes=False),
        scratch_types=[
            pltpu.VMEM((CE,), jnp.int32),
            pltpu.VMEM((n1,), jnp.int32),
            pltpu.VMEM((n1,), jnp.int32),
            pltpu.VMEM((n1,), jnp.int32),
            pltpu.VMEM((2 * n1,), jnp.int32),
            pltpu.VMEM((n2,), jnp.int32),
            pltpu.VMEM((n2,), jnp.int32),
            pltpu.VMEM((n2,), jnp.int32),
            pltpu.SemaphoreType.DMA,
            pltpu.SemaphoreType.DMA,
            pltpu.SemaphoreType.DMA,
        ],
    )
    def sc_kernel(ep_hbm, e2e_hbm, e2ent0_hbm, e2ent1_hbm, e2r_hbm,
                  out_e1, out_r1, out_e2, out_r2,
                  ep_v, idx1_v, e1_v, r1_v, ent_v, idx2_v, e2_v, r2_v,
                  sem, sem2, sem3):
        wid = lax.axis_index("s") * 2 + lax.axis_index("c")
        base = wid * CE
        lane = lax.iota(jnp.int32, L)

        def expand(src_ref, dst_ref, n_src, w, table_len):
            pos0 = lane * w
            for i in range(n_src // L):
                chunk = src_ref[pl.ds(i * L, L)]
                for j in range(w):
                    plsc.store_scatter(dst_ref, [pos0 + (i * L * w + j)],
                                       chunk + j * table_len)

        pltpu.sync_copy(ep_hbm.at[pl.ds(base, CE)], ep_v)
        expand(ep_v, idx1_v, CE, NS, N_ENT)
        pltpu.async_copy(e2e_hbm.at[idx1_v], e1_v, sem).wait()
        c1 = pltpu.async_copy(e2r_hbm.at[e1_v], r1_v, sem2)
        c2 = pltpu.async_copy(e2ent0_hbm.at[e1_v], ent_v.at[pl.ds(0, n1)], sem3)
        pltpu.async_copy(e2ent1_hbm.at[e1_v], ent_v.at[pl.ds(n1, n1)], sem).wait()
        c2.wait()
        expand(ent_v, idx2_v, 2 * n1, NS, N_ENT)
        pltpu.async_copy(e2e_hbm.at[idx2_v], e2_v, sem).wait()
        pltpu.async_copy(e2r_hbm.at[e2_v], r2_v, sem).wait()
        c1.wait()
        pltpu.sync_copy(e1_v, out_e1.at[wid])
        pltpu.sync_copy(r1_v, out_r1.at[wid])
        pltpu.sync_copy(e2_v, out_e2.at[wid])
        pltpu.sync_copy(r2_v, out_r2.at[wid])

    return sc_kernel


BB = 128
R = BB * 16


def _tc_body(rel1_r, e1_r, rel2_r, e2_r, te_r,
             w1a_r, w1b_r, w1c_r, b1_r, w2a_r, w2b_r, b2_r, out_r):
    te = te_r[...]
    m0 = (e1_r[...] != te).astype(jnp.float32)
    m1 = (e2_r[...] != te).astype(jnp.float32)
    rel2 = rel2_r[...]
    iota = lax.broadcasted_iota(jnp.int32, (R, N_REL), 1)
    x_self = (iota == rel1_r[...]).astype(jnp.float32)
    acc0 = jnp.zeros((R, N_REL), jnp.float32)
    acc1 = jnp.zeros((R, N_REL), jnp.float32)
    for c in range(NS):
        acc0 += (iota == rel2[:, c:c + 1]).astype(jnp.float32) * m1[:, c:c + 1]
    for c in range(NS, 2 * NS):
        acc1 += (iota == rel2[:, c:c + 1]).astype(jnp.float32) * m1[:, c:c + 1]
    pre = (jnp.dot(x_self, w1a_r[...], preferred_element_type=jnp.float32)
           + 0.125 * jnp.dot(acc0, w1b_r[...], preferred_element_type=jnp.float32)
           + 0.125 * jnp.dot(acc1, w1c_r[...], preferred_element_type=jnp.float32)
           + b1_r[...])
    v1 = jnp.maximum(pre, 0.0)
    v1m = (v1 * m0 * 0.125).reshape(BB, 2, NS, HIDDEN)
    h = jnp.sum(v1m, axis=2)
    out_r[...] = (jnp.dot(h[:, 0, :], w2a_r[...], preferred_element_type=jnp.float32)
                  + jnp.dot(h[:, 1, :], w2b_r[...], preferred_element_type=jnp.float32)
                  + b2_r[...])


def _tc_compute(rel1_rows, e1_rows, rel2_rows, e2_rows, te_rows,
                W1a, W1b, W1c, b1, W2a, W2b, b2):
    n_rows = B * 16
    grid = (B // BB,)
    full = lambda shape: pl.BlockSpec(shape, lambda g: (0, 0))
    rows = lambda w: pl.BlockSpec((R, w), lambda g: (g, 0))
    return pl.pallas_call(
        _tc_body,
        grid=grid,
        in_specs=[
            rows(1),
            rows(1),
            rows(16),
            rows(16),
            rows(1),
            full((N_REL, HIDDEN)),
            full((N_REL, HIDDEN)),
            full((N_REL, HIDDEN)),
            full((1, HIDDEN)),
            full((HIDDEN, N_REL)),
            full((HIDDEN, N_REL)),
            full((1, N_REL)),
        ],
        out_specs=pl.BlockSpec((BB, N_REL), lambda g: (g, 0)),
        out_shape=jax.ShapeDtypeStruct((B, N_REL), jnp.float32),
    )(rel1_rows.reshape(n_rows, 1), e1_rows.reshape(n_rows, 1),
      rel2_rows, e2_rows, te_rows,
      W1a, W1b, W1c, b1.reshape(1, HIDDEN), W2a, W2b, b2.reshape(1, N_REL))


def kernel(entity_pairs, train_edges, labels, entity2edges, edge2entities,
           edge2relation, relation_features, W1, b1, W2, b2):
    del labels, relation_features
    ep_flat = entity_pairs.reshape(-1)

    e2e_cm = lax.slice(edge2relation, (0,), (N_EDGE,), (4,))
    e2ent0 = edge2entities[:, 0]
    e2ent1 = edge2entities[:, 1]

    sc = _sc_gather_chain()
    out_e1, out_r1, out_e2, out_r2 = sc(
        ep_flat, e2e_cm, e2ent0, e2ent1, edge2relation)

    e1_rows = out_e1.reshape(B, 16)
    r1_rows = out_r1.reshape(B, 16)
    e2_rows = (out_e2.reshape(NW, 2, 512, NS).transpose(0, 2, 1, 3)
               .reshape(B * 16, 16))
    r2_rows = (out_r2.reshape(NW, 2, 512, NS).transpose(0, 2, 1, 3)
               .reshape(B * 16, 16))
    te_rows = jnp.broadcast_to(train_edges[:, None, None], (B, 16, 1)).reshape(B * 16, 1)

    W1a, W1b, W1c = W1[:N_REL], W1[N_REL:2 * N_REL], W1[2 * N_REL:]
    W2a, W2b = W2[:HIDDEN], W2[HIDDEN:]
    return _tc_compute(r1_rows, e1_rows, r2_rows, e2_rows, te_rows,
                       W1a, W1b, W1c, b1, W2a, W2b, b2)


# speedup vs baseline: 3.6570x; 3.6570x over previous
import functools

import jax
import jax.numpy as jnp
from jax import lax
from jax.experimental import pallas as pl
from jax.experimental.pallas import tpu as pltpu
from jax.experimental.pallas import tpu_sc as plsc

B = 1024
N_REL = 237
HIDDEN = 64
NS = 8
N_ENT = 100000
N_EDGE = 3200000

NW = 32
CB = B // NW
CE = CB * 2


def _sc_gather_chain():
    mesh = plsc.VectorSubcoreMesh(core_axis_name="c", subcore_axis_name="s")

    n1 = CE * NS
    n2 = n1 * 2 * NS
    L = 16

    @functools.partial(
        pl.kernel,
        out_type=[
            jax.ShapeDtypeStruct((NW, n1), jnp.int32),
            jax.ShapeDtypeStruct((NW, n1), jnp.int32),
            jax.ShapeDtypeStruct((NW, n2), jnp.int32),
            jax.ShapeDtypeStruct((NW, n2), jnp.int32),
        ],
        mesh=mesh,
        compiler_params=pltpu.CompilerParams(needs_layout_passes=False),
        scratch_types=[
            pltpu.VMEM((CE,), jnp.int32),
            pltpu.VMEM((n1,), jnp.int32),
            pltpu.VMEM((n1,), jnp.int32),
            pltpu.VMEM((n1,), jnp.int32),
            pltpu.VMEM((2 * n1,), jnp.int32),
            pltpu.VMEM((n2,), jnp.int32),
            pltpu.VMEM((n2,), jnp.int32),
            pltpu.VMEM((n2,), jnp.int32),
            pltpu.SemaphoreType.DMA,
            pltpu.SemaphoreType.DMA,
            pltpu.SemaphoreType.DMA,
        ],
    )
    def sc_kernel(ep_hbm, e2e_hbm, e2ent0_hbm, e2ent1_hbm, e2r_hbm,
                  out_e1, out_r1, out_e2, out_r2,
                  ep_v, idx1_v, e1_v, r1_v, ent_v, idx2_v, e2_v, r2_v,
                  sem, sem2, sem3):
        wid = lax.axis_index("s") * 2 + lax.axis_index("c")
        base = wid * CE
        lane = lax.iota(jnp.int32, L)

        def expand(src_ref, dst_ref, n_src, w, table_len):
            pos0 = lane * w
            for i in range(n_src // L):
                chunk = src_ref[pl.ds(i * L, L)]
                for j in range(w):
                    plsc.store_scatter(dst_ref, [pos0 + (i * L * w + j)],
                                       chunk + j * table_len)

        pltpu.sync_copy(ep_hbm.at[pl.ds(base, CE)], ep_v)
        expand(ep_v, idx1_v, CE, NS, N_ENT)
        pltpu.async_copy(e2e_hbm.at[idx1_v], e1_v, sem).wait()
        c1 = pltpu.async_copy(e2r_hbm.at[e1_v], r1_v, sem2)
        c2 = pltpu.async_copy(e2ent0_hbm.at[e1_v], ent_v.at[pl.ds(0, n1)], sem3)
        pltpu.async_copy(e2ent1_hbm.at[e1_v], ent_v.at[pl.ds(n1, n1)], sem).wait()
        c2.wait()
        expand(ent_v, idx2_v, 2 * n1, NS, N_ENT)
        pltpu.async_copy(e2e_hbm.at[idx2_v], e2_v, sem).wait()
        pltpu.async_copy(e2r_hbm.at[e2_v], r2_v, sem).wait()
        c1.wait()
        pltpu.sync_copy(e1_v, out_e1.at[wid])
        pltpu.sync_copy(r1_v, out_r1.at[wid])
        pltpu.sync_copy(e2_v, out_e2.at[wid])
        pltpu.sync_copy(r2_v, out_r2.at[wid])

    return sc_kernel


BB = 128
R = BB * 16


def _tc_body(rel1_r, e1_r, rel2_r, e2_r, te_r,
             w1a_r, w1b_r, w1c_r, b1_r, w2a_r, w2b_r, b2_r, out_r):
    te = te_r[...]
    m0 = (e1_r[...] != te).astype(jnp.float32)
    m1 = (e2_r[...] != te).astype(jnp.float32)
    rel2 = rel2_r[...]
    iota = lax.broadcasted_iota(jnp.int32, (R, N_REL), 1)
    x_self = (iota == rel1_r[...]).astype(jnp.float32)
    acc0 = jnp.zeros((R, N_REL), jnp.float32)
    acc1 = jnp.zeros((R, N_REL), jnp.float32)
    for c in range(NS):
        acc0 += (iota == rel2[:, c:c + 1]).astype(jnp.float32) * m1[:, c:c + 1]
    for c in range(NS, 2 * NS):
        acc1 += (iota == rel2[:, c:c + 1]).astype(jnp.float32) * m1[:, c:c + 1]
    pre = (jnp.dot(x_self, w1a_r[...], preferred_element_type=jnp.float32)
           + 0.125 * jnp.dot(acc0, w1b_r[...], preferred_element_type=jnp.float32)
           + 0.125 * jnp.dot(acc1, w1c_r[...], preferred_element_type=jnp.float32)
           + b1_r[...])
    v1 = jnp.maximum(pre, 0.0)
    v1m = (v1 * m0 * 0.125).reshape(BB, 2, NS, HIDDEN)
    h = jnp.sum(v1m, axis=2)
    out_r[...] = (jnp.dot(h[:, 0, :], w2a_r[...], preferred_element_type=jnp.float32)
                  + jnp.dot(h[:, 1, :], w2b_r[...], preferred_element_type=jnp.float32)
                  + b2_r[...])


def _tc_compute(rel1_rows, e1_rows, rel2_rows, e2_rows, te_rows,
                W1a, W1b, W1c, b1, W2a, W2b, b2):
    n_rows = B * 16
    grid = (B // BB,)
    full = lambda shape: pl.BlockSpec(shape, lambda g: (0, 0))
    rows = lambda w: pl.BlockSpec((R, w), lambda g: (g, 0))
    return pl.pallas_call(
        _tc_body,
        grid=grid,
        in_specs=[
            rows(1),
            rows(1),
            rows(16),
            rows(16),
            rows(1),
            full((N_REL, HIDDEN)),
            full((N_REL, HIDDEN)),
            full((N_REL, HIDDEN)),
            full((1, HIDDEN)),
            full((HIDDEN, N_REL)),
            full((HIDDEN, N_REL)),
            full((1, N_REL)),
        ],
        out_specs=pl.BlockSpec((BB, N_REL), lambda g: (g, 0)),
        out_shape=jax.ShapeDtypeStruct((B, N_REL), jnp.float32),
    )(rel1_rows.reshape(n_rows, 1), e1_rows.reshape(n_rows, 1),
      rel2_rows, e2_rows, te_rows,
      W1a, W1b, W1c, b1.reshape(1, HIDDEN), W2a, W2b, b2.reshape(1, N_REL))


def kernel(entity_pairs, train_edges, labels, entity2edges, edge2entities,
           edge2relation, relation_features, W1, b1, W2, b2):
    del labels, relation_features
    ep_flat = entity_pairs.reshape(-1)

    e2e_cm = entity2edges.T.reshape(-1)
    e2ent0 = edge2entities[:, 0]
    e2ent1 = edge2entities[:, 1]

    sc = _sc_gather_chain()
    out_e1, out_r1, out_e2, out_r2 = sc(
        ep_flat, e2e_cm, e2ent0, e2ent1, edge2relation)

    e1_rows = out_e1.reshape(B, 16)
    r1_rows = out_r1.reshape(B, 16)
    e2_rows = (out_e2.reshape(NW, 2, 512, NS).transpose(0, 2, 1, 3)
               .reshape(B * 16, 16))
    r2_rows = (out_r2.reshape(NW, 2, 512, NS).transpose(0, 2, 1, 3)
               .reshape(B * 16, 16))
    te_rows = jnp.broadcast_to(train_edges[:, None, None], (B, 16, 1)).reshape(B * 16, 1)

    W1a, W1b, W1c = W1[:N_REL], W1[N_REL:2 * N_REL], W1[2 * N_REL:]
    W2a, W2b = W2[:HIDDEN], W2[HIDDEN:]
    return _tc_compute(r1_rows, e1_rows, r2_rows, e2_rows, te_rows,
                       W1a, W1b, W1c, b1, W2a, W2b, b2)


# speedup vs baseline: 3.6939x; 1.0101x over previous
import functools

import jax
import jax.numpy as jnp
from jax import lax
from jax.experimental import pallas as pl
from jax.experimental.pallas import tpu as pltpu
from jax.experimental.pallas import tpu_sc as plsc

B = 1024
N_REL = 237
HIDDEN = 64
NS = 8
N_ENT = 100000
N_EDGE = 3200000

NW = 32
CB = B // NW
CE = CB * 2


def _sc_gather_chain():
    mesh = plsc.VectorSubcoreMesh(core_axis_name="c", subcore_axis_name="s")

    n1 = CE * NS
    n2 = n1 * 2 * NS
    L = 16

    @functools.partial(
        pl.kernel,
        out_type=[
            jax.ShapeDtypeStruct((NW, n1), jnp.int32),
            jax.ShapeDtypeStruct((NW, n1), jnp.int32),
            jax.ShapeDtypeStruct((NW, n2), jnp.int32),
            jax.ShapeDtypeStruct((NW, n2), jnp.int32),
        ],
        mesh=mesh,
        compiler_params=pltpu.CompilerParams(needs_layout_passes=False),
        scratch_types=[
            pltpu.VMEM((CE,), jnp.int32),
            pltpu.VMEM((n1,), jnp.int32),
            pltpu.VMEM((n1,), jnp.int32),
            pltpu.VMEM((n1,), jnp.int32),
            pltpu.VMEM((2 * n1,), jnp.int32),
            pltpu.VMEM((n2,), jnp.int32),
            pltpu.VMEM((n2,), jnp.int32),
            pltpu.VMEM((n2,), jnp.int32),
            pltpu.SemaphoreType.DMA,
            pltpu.SemaphoreType.DMA,
            pltpu.SemaphoreType.DMA,
        ],
    )
    def sc_kernel(ep_hbm, e2e_hbm, e2ent0_hbm, e2ent1_hbm, e2r_hbm,
                  out_e1, out_r1, out_e2, out_r2,
                  ep_v, idx1_v, e1_v, r1_v, ent_v, idx2_v, e2_v, r2_v,
                  sem, sem2, sem3):
        wid = lax.axis_index("s") * 2 + lax.axis_index("c")
        base = wid * CE
        lane = lax.iota(jnp.int32, L)

        def expand(src_ref, dst_ref, n_src, w, table_len):
            pos0 = lane * w
            for i in range(n_src // L):
                chunk = src_ref[pl.ds(i * L, L)]
                for j in range(w):
                    plsc.store_scatter(dst_ref, [pos0 + (i * L * w + j)],
                                       chunk + j * table_len)

        pltpu.sync_copy(ep_hbm.at[pl.ds(base, CE)], ep_v)
        expand(ep_v, idx1_v, CE, NS, N_ENT)
        pltpu.async_copy(e2e_hbm.at[idx1_v], e1_v, sem).wait()
        c1 = pltpu.async_copy(e2r_hbm.at[e1_v], r1_v, sem2)
        c2 = pltpu.async_copy(e2ent0_hbm.at[e1_v], ent_v.at[pl.ds(0, n1)], sem3)
        pltpu.async_copy(e2ent1_hbm.at[e1_v], ent_v.at[pl.ds(n1, n1)], sem).wait()
        c2.wait()
        expand(ent_v, idx2_v, 2 * n1, NS, N_ENT)
        pltpu.async_copy(e2e_hbm.at[idx2_v], e2_v, sem).wait()
        pltpu.async_copy(e2r_hbm.at[e2_v], r2_v, sem).wait()
        c1.wait()
        pltpu.sync_copy(e1_v, out_e1.at[wid])
        pltpu.sync_copy(r1_v, out_r1.at[wid])
        pltpu.sync_copy(e2_v, out_e2.at[wid])
        pltpu.sync_copy(r2_v, out_r2.at[wid])

    return sc_kernel


BB = 128
R = BB * 16


def _tc_body(rel1_r, e1_r, rel2_r, e2_r, te_r,
             w1a_r, w1b_r, w1c_r, b1_r, w2a_r, w2b_r, b2_r, out_r):
    te_b = te_r[...]
    te = jnp.broadcast_to(te_b[:, None, :], (BB, 16, 1)).reshape(R, 1)
    m0 = (e1_r[...] != te).astype(jnp.float32)
    m1 = (e2_r[...] != te).astype(jnp.float32)
    rel2 = rel2_r[...]
    iota = lax.broadcasted_iota(jnp.int32, (R, N_REL), 1)
    x_self = (iota == rel1_r[...]).astype(jnp.float32)
    acc0 = jnp.zeros((R, N_REL), jnp.float32)
    acc1 = jnp.zeros((R, N_REL), jnp.float32)
    for c in range(NS):
        acc0 += (iota == rel2[:, c:c + 1]).astype(jnp.float32) * m1[:, c:c + 1]
    for c in range(NS, 2 * NS):
        acc1 += (iota == rel2[:, c:c + 1]).astype(jnp.float32) * m1[:, c:c + 1]
    pre = (jnp.dot(x_self, w1a_r[...], preferred_element_type=jnp.float32)
           + 0.125 * jnp.dot(acc0, w1b_r[...], preferred_element_type=jnp.float32)
           + 0.125 * jnp.dot(acc1, w1c_r[...], preferred_element_type=jnp.float32)
           + b1_r[...])
    v1 = jnp.maximum(pre, 0.0)
    v1m = (v1 * m0 * 0.125).reshape(BB, 2, NS, HIDDEN)
    h = jnp.sum(v1m, axis=2)
    out_r[...] = (jnp.dot(h[:, 0, :], w2a_r[...], preferred_element_type=jnp.float32)
                  + jnp.dot(h[:, 1, :], w2b_r[...], preferred_element_type=jnp.float32)
                  + b2_r[...])


def _tc_compute(rel1_rows, e1_rows, rel2_rows, e2_rows, te_col,
                W1a, W1b, W1c, b1, W2a, W2b, b2):
    n_rows = B * 16
    grid = (B // BB,)
    full = lambda shape: pl.BlockSpec(shape, lambda g: (0, 0))
    rows = lambda w: pl.BlockSpec((R, w), lambda g: (g, 0))
    return pl.pallas_call(
        _tc_body,
        grid=grid,
        in_specs=[
            rows(1),
            rows(1),
            rows(16),
            rows(16),
            pl.BlockSpec((BB, 1), lambda g: (g, 0)),
            full((N_REL, HIDDEN)),
            full((N_REL, HIDDEN)),
            full((N_REL, HIDDEN)),
            full((1, HIDDEN)),
            full((HIDDEN, N_REL)),
            full((HIDDEN, N_REL)),
            full((1, N_REL)),
        ],
        out_specs=pl.BlockSpec((BB, N_REL), lambda g: (g, 0)),
        out_shape=jax.ShapeDtypeStruct((B, N_REL), jnp.float32),
    )(rel1_rows.reshape(n_rows, 1), e1_rows.reshape(n_rows, 1),
      rel2_rows, e2_rows, te_col,
      W1a, W1b, W1c, b1.reshape(1, HIDDEN), W2a, W2b, b2.reshape(1, N_REL))


def kernel(entity_pairs, train_edges, labels, entity2edges, edge2entities,
           edge2relation, relation_features, W1, b1, W2, b2):
    del labels, relation_features
    ep_flat = entity_pairs.reshape(-1)

    e2e_cm = entity2edges.T.reshape(-1)
    e2ent0 = edge2entities[:, 0]
    e2ent1 = edge2entities[:, 1]

    sc = _sc_gather_chain()
    out_e1, out_r1, out_e2, out_r2 = sc(
        ep_flat, e2e_cm, e2ent0, e2ent1, edge2relation)

    e1_rows = out_e1.reshape(B, 16)
    r1_rows = out_r1.reshape(B, 16)
    e2_rows = (out_e2.reshape(NW, 2, 512, NS).transpose(0, 2, 1, 3)
               .reshape(B * 16, 16))
    r2_rows = (out_r2.reshape(NW, 2, 512, NS).transpose(0, 2, 1, 3)
               .reshape(B * 16, 16))
    te_col = train_edges[:, None]

    W1a, W1b, W1c = W1[:N_REL], W1[N_REL:2 * N_REL], W1[2 * N_REL:]
    W2a, W2b = W2[:HIDDEN], W2[HIDDEN:]
    return _tc_compute(r1_rows, e1_rows, r2_rows, e2_rows, te_col,
                       W1a, W1b, W1c, b1, W2a, W2b, b2)


# speedup vs baseline: 7.0059x; 1.8966x over previous
import functools

import jax
import jax.numpy as jnp
from jax import lax
from jax.experimental import pallas as pl
from jax.experimental.pallas import tpu as pltpu
from jax.experimental.pallas import tpu_sc as plsc

B = 1024
N_REL = 237
HIDDEN = 64
NS = 8
N_ENT = 100000
N_EDGE = 3200000

NW = 32
CB = B // NW
CE = CB * 2


N1 = CE * NS
N2 = N1 * 2 * NS
L = 16

_MESH = plsc.VectorSubcoreMesh(core_axis_name="c", subcore_axis_name="s")
_SC_PARAMS = pltpu.CompilerParams(needs_layout_passes=False)


def _expand(lane, src_ref, dst_ref, n_src, w, table_len):
    pos0 = lane * w
    for i in range(n_src // L):
        chunk = src_ref[pl.ds(i * L, L)]
        for j in range(w):
            plsc.store_scatter(dst_ref, [pos0 + (i * L * w + j)],
                               chunk + j * table_len)


def _sc_hop1():
    @functools.partial(
        pl.kernel,
        out_type=[
            jax.ShapeDtypeStruct((NW, N1), jnp.int32),
            jax.ShapeDtypeStruct((NW, N1), jnp.int32),
        ],
        mesh=_MESH,
        compiler_params=_SC_PARAMS,
        scratch_types=[
            pltpu.VMEM((CE,), jnp.int32),
            pltpu.VMEM((N1,), jnp.int32),
            pltpu.VMEM((N1,), jnp.int32),
            pltpu.VMEM((N1,), jnp.int32),
            pltpu.SemaphoreType.DMA,
        ],
    )
    def hop1(ep_hbm, e2e_hbm, e2r_hbm, out_e1, out_r1,
             ep_v, idx1_v, e1_v, r1_v, sem):
        wid = lax.axis_index("s") * 2 + lax.axis_index("c")
        lane = lax.iota(jnp.int32, L)
        pltpu.sync_copy(ep_hbm.at[pl.ds(wid * CE, CE)], ep_v)
        _expand(lane, ep_v, idx1_v, CE, NS, N_ENT)
        pltpu.async_copy(e2e_hbm.at[idx1_v], e1_v, sem).wait()
        pltpu.async_copy(e2r_hbm.at[e1_v], r1_v, sem).wait()
        pltpu.sync_copy(e1_v, out_e1.at[wid])
        pltpu.sync_copy(r1_v, out_r1.at[wid])

    return hop1


def _sc_hop2():
    @functools.partial(
        pl.kernel,
        out_type=[
            jax.ShapeDtypeStruct((NW, N2), jnp.int32),
            jax.ShapeDtypeStruct((NW, N2), jnp.int32),
        ],
        mesh=_MESH,
        compiler_params=_SC_PARAMS,
        scratch_types=[
            pltpu.VMEM((N1,), jnp.int32),
            pltpu.VMEM((N1,), jnp.int32),
            pltpu.VMEM((2 * N1,), jnp.int32),
            pltpu.VMEM((N2,), jnp.int32),
            pltpu.VMEM((N2,), jnp.int32),
            pltpu.VMEM((N2,), jnp.int32),
            pltpu.SemaphoreType.DMA,
        ],
    )
    def hop2(ents0_hbm, ents1_hbm, e2e_hbm, e2r_hbm, out_e2, out_r2,
             entA_v, entB_v, ent_v, idx2_v, e2_v, r2_v, sem):
        wid = lax.axis_index("s") * 2 + lax.axis_index("c")
        lane = lax.iota(jnp.int32, L)
        base = wid * N1
        pltpu.sync_copy(ents0_hbm.at[pl.ds(base, N1)], entA_v)
        pltpu.sync_copy(ents1_hbm.at[pl.ds(base, N1)], entB_v)
        pos0 = lane * 2
        for i in range(N1 // L):
            plsc.store_scatter(ent_v, [pos0 + i * 2 * L], entA_v[pl.ds(i * L, L)])
            plsc.store_scatter(ent_v, [pos0 + (i * 2 * L + 1)], entB_v[pl.ds(i * L, L)])
        _expand(lane, ent_v, idx2_v, 2 * N1, NS, N_ENT)
        pltpu.async_copy(e2e_hbm.at[idx2_v], e2_v, sem).wait()
        pltpu.async_copy(e2r_hbm.at[e2_v], r2_v, sem).wait()
        pltpu.sync_copy(e2_v, out_e2.at[wid])
        pltpu.sync_copy(r2_v, out_r2.at[wid])

    return hop2


BB = 128
R = BB * 16


def _tc_body(rel1_r, e1_r, rel2_r, e2_r, te_r,
             w1a_r, w1b_r, w1c_r, b1_r, w2a_r, w2b_r, b2_r, out_r):
    te_b = te_r[...]
    te = jnp.broadcast_to(te_b[:, None, :], (BB, 16, 1)).reshape(R, 1)
    m0 = (e1_r[...] != te).astype(jnp.float32)
    m1 = (e2_r[...] != te).astype(jnp.float32)
    rel2 = rel2_r[...]
    iota = lax.broadcasted_iota(jnp.int32, (R, N_REL), 1)
    x_self = (iota == rel1_r[...]).astype(jnp.float32)
    acc0 = jnp.zeros((R, N_REL), jnp.float32)
    acc1 = jnp.zeros((R, N_REL), jnp.float32)
    for c in range(NS):
        acc0 += (iota == rel2[:, c:c + 1]).astype(jnp.float32) * m1[:, c:c + 1]
    for c in range(NS, 2 * NS):
        acc1 += (iota == rel2[:, c:c + 1]).astype(jnp.float32) * m1[:, c:c + 1]
    pre = (jnp.dot(x_self, w1a_r[...], preferred_element_type=jnp.float32)
           + 0.125 * jnp.dot(acc0, w1b_r[...], preferred_element_type=jnp.float32)
           + 0.125 * jnp.dot(acc1, w1c_r[...], preferred_element_type=jnp.float32)
           + b1_r[...])
    v1 = jnp.maximum(pre, 0.0)
    v1m = (v1 * m0 * 0.125).reshape(BB, 2, NS, HIDDEN)
    h = jnp.sum(v1m, axis=2)
    out_r[...] = (jnp.dot(h[:, 0, :], w2a_r[...], preferred_element_type=jnp.float32)
                  + jnp.dot(h[:, 1, :], w2b_r[...], preferred_element_type=jnp.float32)
                  + b2_r[...])


def _tc_compute(rel1_rows, e1_rows, rel2_rows, e2_rows, te_col,
                W1a, W1b, W1c, b1, W2a, W2b, b2):
    n_rows = B * 16
    grid = (B // BB,)
    full = lambda shape: pl.BlockSpec(shape, lambda g: (0, 0))
    rows = lambda w: pl.BlockSpec((R, w), lambda g: (g, 0))
    return pl.pallas_call(
        _tc_body,
        grid=grid,
        in_specs=[
            rows(1),
            rows(1),
            rows(16),
            rows(16),
            pl.BlockSpec((BB, 1), lambda g: (g, 0)),
            full((N_REL, HIDDEN)),
            full((N_REL, HIDDEN)),
            full((N_REL, HIDDEN)),
            full((1, HIDDEN)),
            full((HIDDEN, N_REL)),
            full((HIDDEN, N_REL)),
            full((1, N_REL)),
        ],
        out_specs=pl.BlockSpec((BB, N_REL), lambda g: (g, 0)),
        out_shape=jax.ShapeDtypeStruct((B, N_REL), jnp.float32),
    )(rel1_rows.reshape(n_rows, 1), e1_rows.reshape(n_rows, 1),
      rel2_rows, e2_rows, te_col,
      W1a, W1b, W1c, b1.reshape(1, HIDDEN), W2a, W2b, b2.reshape(1, N_REL))


def kernel(entity_pairs, train_edges, labels, entity2edges, edge2entities,
           edge2relation, relation_features, W1, b1, W2, b2):
    del labels, relation_features
    ep_flat = entity_pairs.reshape(-1)

    e2e_cm = entity2edges.T.reshape(-1)

    out_e1, out_r1 = _sc_hop1()(ep_flat, e2e_cm, edge2relation)
    ents = jnp.take(edge2entities, out_e1.reshape(-1), axis=0)
    out_e2, out_r2 = _sc_hop2()(ents[:, 0], ents[:, 1], e2e_cm, edge2relation)

    e1_rows = out_e1.reshape(B, 16)
    r1_rows = out_r1.reshape(B, 16)
    e2_rows = out_e2.reshape(B * 16, 16)
    r2_rows = out_r2.reshape(B * 16, 16)
    te_col = train_edges[:, None]

    W1a, W1b, W1c = W1[:N_REL], W1[N_REL:2 * N_REL], W1[2 * N_REL:]
    W2a, W2b = W2[:HIDDEN], W2[HIDDEN:]
    return _tc_compute(r1_rows, e1_rows, r2_rows, e2_rows, te_col,
                       W1a, W1b, W1c, b1, W2a, W2b, b2)


# speedup vs baseline: 7.3156x; 1.0442x over previous
import functools

import jax
import jax.numpy as jnp
from jax import lax
from jax.experimental import pallas as pl
from jax.experimental.pallas import tpu as pltpu
from jax.experimental.pallas import tpu_sc as plsc

B = 1024
N_REL = 237
HIDDEN = 64
NS = 8
N_ENT = 100000
N_EDGE = 3200000

NW = 32
CB = B // NW
CE = CB * 2


N1 = CE * NS
N2 = N1 * 2 * NS
L = 16

_SC_PARAMS = pltpu.CompilerParams(needs_layout_passes=False)


def _mesh():
    return plsc.VectorSubcoreMesh(core_axis_name="c", subcore_axis_name="s")


def _expand(lane, src_ref, dst_ref, n_src, w, table_len):
    pos0 = lane * w
    for i in range(n_src // L):
        chunk = src_ref[pl.ds(i * L, L)]
        for j in range(w):
            plsc.store_scatter(dst_ref, [pos0 + (i * L * w + j)],
                               chunk + j * table_len)


def _sc_hop1():
    @functools.partial(
        pl.kernel,
        out_type=[
            jax.ShapeDtypeStruct((NW, N1), jnp.int32),
            jax.ShapeDtypeStruct((NW, N1), jnp.int32),
        ],
        mesh=_mesh(),
        compiler_params=_SC_PARAMS,
        scratch_types=[
            pltpu.VMEM((CE,), jnp.int32),
            pltpu.VMEM((N1,), jnp.int32),
            pltpu.VMEM((N1,), jnp.int32),
            pltpu.VMEM((N1,), jnp.int32),
            pltpu.SemaphoreType.DMA,
        ],
    )
    def hop1(ep_hbm, e2e_hbm, e2r_hbm, out_e1, out_r1,
             ep_v, idx1_v, e1_v, r1_v, sem):
        wid = lax.axis_index("s") * 2 + lax.axis_index("c")
        lane = lax.iota(jnp.int32, L)
        pltpu.sync_copy(ep_hbm.at[pl.ds(wid * CE, CE)], ep_v)
        _expand(lane, ep_v, idx1_v, CE, NS, N_ENT)
        pltpu.async_copy(e2e_hbm.at[idx1_v], e1_v, sem).wait()
        pltpu.async_copy(e2r_hbm.at[e1_v], r1_v, sem).wait()
        pltpu.sync_copy(e1_v, out_e1.at[wid])
        pltpu.sync_copy(r1_v, out_r1.at[wid])

    return hop1


def _sc_hop2():
    @functools.partial(
        pl.kernel,
        out_type=[
            jax.ShapeDtypeStruct((NW, N2), jnp.int32),
            jax.ShapeDtypeStruct((NW, N2), jnp.int32),
        ],
        mesh=_mesh(),
        compiler_params=_SC_PARAMS,
        scratch_types=[
            pltpu.VMEM((N1,), jnp.int32),
            pltpu.VMEM((N1,), jnp.int32),
            pltpu.VMEM((2 * N1,), jnp.int32),
            pltpu.VMEM((N2,), jnp.int32),
            pltpu.VMEM((N2,), jnp.int32),
            pltpu.VMEM((N2,), jnp.int32),
            pltpu.SemaphoreType.DMA,
        ],
    )
    def hop2(ents0_hbm, ents1_hbm, e2e_hbm, e2r_hbm, out_e2, out_r2,
             entA_v, entB_v, ent_v, idx2_v, e2_v, r2_v, sem):
        wid = lax.axis_index("s") * 2 + lax.axis_index("c")
        lane = lax.iota(jnp.int32, L)
        base = wid * N1
        pltpu.sync_copy(ents0_hbm.at[pl.ds(base, N1)], entA_v)
        pltpu.sync_copy(ents1_hbm.at[pl.ds(base, N1)], entB_v)
        pos0 = lane * 2
        for i in range(N1 // L):
            plsc.store_scatter(ent_v, [pos0 + i * 2 * L], entA_v[pl.ds(i * L, L)])
            plsc.store_scatter(ent_v, [pos0 + (i * 2 * L + 1)], entB_v[pl.ds(i * L, L)])
        _expand(lane, ent_v, idx2_v, 2 * N1, NS, N_ENT)
        pltpu.async_copy(e2e_hbm.at[idx2_v], e2_v, sem).wait()
        pltpu.async_copy(e2r_hbm.at[e2_v], r2_v, sem).wait()
        pltpu.sync_copy(e2_v, out_e2.at[wid])
        pltpu.sync_copy(r2_v, out_r2.at[wid])

    return hop2


BB = 128
R = BB * 16


def _tc_body(rel1_r, e1_r, rel2_r, e2_r, te_r,
             w1a_r, w1b_r, w1c_r, b1_r, w2a_r, w2b_r, b2_r, out_r):
    te_b = te_r[...]
    te = jnp.broadcast_to(te_b[:, None, :], (BB, 16, 1)).reshape(R, 1)
    m0 = (e1_r[...] != te).astype(jnp.float32)
    m1 = (e2_r[...] != te).astype(jnp.float32)
    rel2 = rel2_r[...]
    iota = lax.broadcasted_iota(jnp.int32, (R, N_REL), 1)
    x_self = (iota == rel1_r[...]).astype(jnp.float32)
    acc0 = jnp.zeros((R, N_REL), jnp.float32)
    acc1 = jnp.zeros((R, N_REL), jnp.float32)
    for c in range(NS):
        acc0 += jnp.where(iota == rel2[:, c:c + 1], m1[:, c:c + 1], 0.0)
    for c in range(NS, 2 * NS):
        acc1 += jnp.where(iota == rel2[:, c:c + 1], m1[:, c:c + 1], 0.0)
    pre = (jnp.dot(x_self, w1a_r[...], preferred_element_type=jnp.float32)
           + jnp.dot(acc0, w1b_r[...], preferred_element_type=jnp.float32)
           + jnp.dot(acc1, w1c_r[...], preferred_element_type=jnp.float32)
           + b1_r[...])
    v1 = jnp.maximum(pre, 0.0)
    v1m = (v1 * m0 * 0.125).reshape(BB, 2, NS, HIDDEN)
    h = jnp.sum(v1m, axis=2)
    out_r[...] = (jnp.dot(h[:, 0, :], w2a_r[...], preferred_element_type=jnp.float32)
                  + jnp.dot(h[:, 1, :], w2b_r[...], preferred_element_type=jnp.float32)
                  + b2_r[...])


def _tc_compute(rel1_rows, e1_rows, rel2_rows, e2_rows, te_col,
                W1a, W1b, W1c, b1, W2a, W2b, b2):
    n_rows = B * 16
    grid = (B // BB,)
    full = lambda shape: pl.BlockSpec(shape, lambda g: (0, 0))
    rows = lambda w: pl.BlockSpec((R, w), lambda g: (g, 0))
    return pl.pallas_call(
        _tc_body,
        grid=grid,
        in_specs=[
            rows(1),
            rows(1),
            rows(16),
            rows(16),
            pl.BlockSpec((BB, 1), lambda g: (g, 0)),
            full((N_REL, HIDDEN)),
            full((N_REL, HIDDEN)),
            full((N_REL, HIDDEN)),
            full((1, HIDDEN)),
            full((HIDDEN, N_REL)),
            full((HIDDEN, N_REL)),
            full((1, N_REL)),
        ],
        out_specs=pl.BlockSpec((BB, N_REL), lambda g: (g, 0)),
        out_shape=jax.ShapeDtypeStruct((B, N_REL), jnp.float32),
    )(rel1_rows.reshape(n_rows, 1), e1_rows.reshape(n_rows, 1),
      rel2_rows, e2_rows, te_col,
      W1a, W1b, W1c, b1.reshape(1, HIDDEN), W2a, W2b, b2.reshape(1, N_REL))


def kernel(entity_pairs, train_edges, labels, entity2edges, edge2entities,
           edge2relation, relation_features, W1, b1, W2, b2):
    del labels, relation_features
    ep_flat = entity_pairs.reshape(-1)

    e2e_cm = entity2edges.T.reshape(-1)

    out_e1, out_r1 = _sc_hop1()(ep_flat, e2e_cm, edge2relation)
    ents = jnp.take(edge2entities, out_e1.reshape(-1), axis=0)
    out_e2, out_r2 = _sc_hop2()(ents[:, 0], ents[:, 1], e2e_cm, edge2relation)

    e1_rows = out_e1.reshape(B, 16)
    r1_rows = out_r1.reshape(B, 16)
    e2_rows = out_e2.reshape(B * 16, 16)
    r2_rows = out_r2.reshape(B * 16, 16)
    te_col = train_edges[:, None]

    W1a = W1[:N_REL]
    W1b = W1[N_REL:2 * N_REL] * 0.125
    W1c = W1[2 * N_REL:] * 0.125
    W2a, W2b = W2[:HIDDEN], W2[HIDDEN:]
    return _tc_compute(r1_rows, e1_rows, r2_rows, e2_rows, te_col,
                       W1a, W1b, W1c, b1, W2a, W2b, b2)
